# baseline (speedup 1.0000x reference)
import jax
import jax.numpy as jnp
from jax.experimental import pallas as pl
from jax.experimental.pallas import tpu as pltpu


def _body(idx_ref, vid_ref, ft_ref, pos_ref, out_ref):
    out_ref[...] = vid_ref[...] + ft_ref[...] + pos_ref[...]


def kernel(video_embeds, video_frame_mask, frame_token, pos_token):
    N, P, D = video_embeds.shape
    frame_idx = (
        jnp.cumsum(video_frame_mask.astype(jnp.int32), axis=-1) - 1
    ).reshape(-1)
    grid_spec = pltpu.PrefetchScalarGridSpec(
        num_scalar_prefetch=1,
        grid=(N,),
        in_specs=[
            pl.BlockSpec((1, P, D), lambda i, idx: (i, 0, 0)),
            pl.BlockSpec((1, 1, D), lambda i, idx: (idx[i], 0, 0)),
            pl.BlockSpec((1, P, D), lambda i, idx: (0, 0, 0)),
        ],
        out_specs=pl.BlockSpec((1, P, D), lambda i, idx: (i, 0, 0)),
    )
    return pl.pallas_call(
        _body,
        grid_spec=grid_spec,
        out_shape=jax.ShapeDtypeStruct((N, P, D), video_embeds.dtype),
    )(frame_idx, video_embeds, frame_token, pos_token)


# speedup vs baseline: 2.1720x; 2.1720x over previous
import jax
import jax.numpy as jnp
from jax.experimental import pallas as pl
from jax.experimental.pallas import tpu as pltpu

_F = 8


def _body(idx_ref, vid_ref, ft_ref, pos_ref, out_ref):
    v = pl.program_id(0)
    for f in range(_F):
        row = idx_ref[v, f]
        fts = ft_ref[pl.ds(row, 1), 0, :]
        out_ref[f] = vid_ref[f] + fts + pos_ref[0]


def kernel(video_embeds, video_frame_mask, frame_token, pos_token):
    N, P, D = video_embeds.shape
    B, F = video_frame_mask.shape
    frame_idx = (jnp.cumsum(video_frame_mask.astype(jnp.int32), axis=-1) - 1)
    grid_spec = pltpu.PrefetchScalarGridSpec(
        num_scalar_prefetch=1,
        grid=(B,),
        in_specs=[
            pl.BlockSpec((F, P, D), lambda i, idx: (i, 0, 0)),
            pl.BlockSpec((F, 1, D), lambda i, idx: (0, 0, 0)),
            pl.BlockSpec((1, P, D), lambda i, idx: (0, 0, 0)),
        ],
        out_specs=pl.BlockSpec((F, P, D), lambda i, idx: (i, 0, 0)),
    )
    return pl.pallas_call(
        _body,
        grid_spec=grid_spec,
        out_shape=jax.ShapeDtypeStruct((N, P, D), video_embeds.dtype),
    )(frame_idx, video_embeds, frame_token, pos_token)


# speedup vs baseline: 2.2217x; 1.0229x over previous
import jax
import jax.numpy as jnp
from jax.experimental import pallas as pl
from jax.experimental.pallas import tpu as pltpu

_R = 16


def _body(idx_ref, vid_ref, ft_ref, pos_ref, out_ref):
    s = pl.program_id(0)
    for j in range(_R):
        row = idx_ref[s * _R + j]
        fts = ft_ref[pl.ds(row, 1), 0, :]
        out_ref[j] = vid_ref[j] + fts + pos_ref[0]


def kernel(video_embeds, video_frame_mask, frame_token, pos_token):
    N, P, D = video_embeds.shape
    F = video_frame_mask.shape[-1]
    frame_idx = (
        jnp.cumsum(video_frame_mask.astype(jnp.int32), axis=-1) - 1
    ).reshape(-1)
    grid_spec = pltpu.PrefetchScalarGridSpec(
        num_scalar_prefetch=1,
        grid=(N // _R,),
        in_specs=[
            pl.BlockSpec((_R, P, D), lambda i, idx: (i, 0, 0)),
            pl.BlockSpec((F, 1, D), lambda i, idx: (0, 0, 0)),
            pl.BlockSpec((1, P, D), lambda i, idx: (0, 0, 0)),
        ],
        out_specs=pl.BlockSpec((_R, P, D), lambda i, idx: (i, 0, 0)),
    )
    return pl.pallas_call(
        _body,
        grid_spec=grid_spec,
        out_shape=jax.ShapeDtypeStruct((N, P, D), video_embeds.dtype),
    )(frame_idx, video_embeds, frame_token, pos_token)


# speedup vs baseline: 2.2246x; 1.0013x over previous
import jax
import jax.numpy as jnp
from jax.experimental import pallas as pl
from jax.experimental.pallas import tpu as pltpu

_R = 16


def _body(idx_ref, vid_ref, ft_ref, pos_ref, out_ref, comb_ref):
    s = pl.program_id(0)

    @pl.when(s == 0)
    def _build_comb():
        for f in range(ft_ref.shape[0]):
            comb_ref[f] = ft_ref[f] + pos_ref[0]

    for j in range(_R):
        row = idx_ref[s * _R + j]
        out_ref[j] = vid_ref[j] + comb_ref[pl.ds(row, 1)][0]


def kernel(video_embeds, video_frame_mask, frame_token, pos_token):
    N, P, D = video_embeds.shape
    F = video_frame_mask.shape[-1]
    frame_idx = (
        jnp.cumsum(video_frame_mask.astype(jnp.int32), axis=-1) - 1
    ).reshape(-1)
    grid_spec = pltpu.PrefetchScalarGridSpec(
        num_scalar_prefetch=1,
        grid=(N // _R,),
        in_specs=[
            pl.BlockSpec((_R, P, D), lambda i, idx: (i, 0, 0)),
            pl.BlockSpec((F, 1, D), lambda i, idx: (0, 0, 0)),
            pl.BlockSpec((1, P, D), lambda i, idx: (0, 0, 0)),
        ],
        out_specs=pl.BlockSpec((_R, P, D), lambda i, idx: (i, 0, 0)),
        scratch_shapes=[pltpu.VMEM((F, P, D), jnp.float32)],
    )
    return pl.pallas_call(
        _body,
        grid_spec=grid_spec,
        out_shape=jax.ShapeDtypeStruct((N, P, D), video_embeds.dtype),
    )(frame_idx, video_embeds, frame_token, pos_token)
